# Initial kernel scaffold; baseline (speedup 1.0000x reference)
import jax
import jax.numpy as jnp
from jax.experimental import pallas as pl


def kernel(x, Wqkv, bqkv, Wlepe, blepe, Wout, bout):
    raise NotImplementedError("write your pallas kernel here")


# speedup vs baseline: 1.1952x; 1.1952x over previous
import functools

import jax
import jax.numpy as jnp
from jax.experimental import pallas as pl
from jax.experimental.pallas import tpu as pltpu

_NH = 8
_NWIN = 7
_TOPK = 4


def _qkv_kernel(x_ref, wq_ref, wk_ref, wv_ref, bq_ref, bk_ref, bv_ref,
                q_ref, k_ref, v_ref, pq_ref, pk_ref):
    nwin = q_ref.shape[1]
    rsq = q_ref.shape[2]
    c = q_ref.shape[3]
    xb = x_ref[0].reshape(nwin * rsq, c)
    q = jnp.dot(xb, wq_ref[...], preferred_element_type=jnp.float32) + bq_ref[...]
    k = jnp.dot(xb, wk_ref[...], preferred_element_type=jnp.float32) + bk_ref[...]
    v = jnp.dot(xb, wv_ref[...], preferred_element_type=jnp.float32) + bv_ref[...]
    q3 = q.reshape(nwin, rsq, c)
    k3 = k.reshape(nwin, rsq, c)
    q_ref[0] = q3
    k_ref[0] = k3
    v_ref[0] = v.reshape(nwin, rsq, c)
    pq_ref[0, 0] = jnp.mean(q3, axis=1)
    pk_ref[0, 0] = jnp.mean(k3, axis=1)


def _route_kernel(qr_ref, kr_ref, idx_ref):
    qr = qr_ref[0]
    kr = kr_ref[0]
    a = jax.lax.dot_general(qr, kr, (((1,), (1,)), ((), ())),
                            preferred_element_type=jnp.float32)
    col = jax.lax.broadcasted_iota(jnp.int32, a.shape, 1)
    picks = []
    for _ in range(_TOPK):
        m = jnp.max(a, axis=1, keepdims=True)
        cand = jnp.where(a == m, col, jnp.int32(2 ** 30))
        j = jnp.min(cand, axis=1, keepdims=True)
        picks.append(j)
        a = jnp.where(col == j, -jnp.inf, a)
    idx_ref[0] = jnp.concatenate(picks, axis=1)


def _attn_kernel(idx_ref, q_ref, k0_ref, k1_ref, k2_ref, k3_ref,
                 v0_ref, v1_ref, v2_ref, v3_ref, o_ref, *, scale, nh):
    del idx_ref
    q = q_ref[0, 0] * scale
    k = jnp.concatenate([k0_ref[0, 0], k1_ref[0, 0],
                         k2_ref[0, 0], k3_ref[0, 0]], axis=0)
    v = jnp.concatenate([v0_ref[0, 0], v1_ref[0, 0],
                         v2_ref[0, 0], v3_ref[0, 0]], axis=0)
    hd = q.shape[1] // nh
    outs = []
    for h in range(nh):
        sl = slice(h * hd, (h + 1) * hd)
        qh = q[:, sl]
        kh = k[:, sl]
        vh = v[:, sl]
        s = jax.lax.dot_general(qh, kh, (((1,), (1,)), ((), ())),
                                preferred_element_type=jnp.float32)
        m = jnp.max(s, axis=1, keepdims=True)
        e = jnp.exp(s - m)
        p = e / jnp.sum(e, axis=1, keepdims=True)
        outs.append(jnp.dot(p, vh, preferred_element_type=jnp.float32))
    o_ref[0, 0] = jnp.concatenate(outs, axis=1)


def _out_kernel(ao_ref, vp_ref, wl_ref, bl_ref, wo_ref, bo_ref, o_ref):
    h = ao_ref.shape[1]
    w = ao_ref.shape[2]
    c = ao_ref.shape[3]
    acc = ao_ref[0]
    wl = wl_ref[...]
    for ky in range(3):
        for kx in range(3):
            acc = acc + vp_ref[0, ky:ky + h, kx:kx + w, :] * wl[ky * 3 + kx][None, None, :]
    acc = acc + bl_ref[...][None]
    t = acc.reshape(h * w, c)
    out = jnp.dot(t, wo_ref[...], preferred_element_type=jnp.float32) + bo_ref[...]
    o_ref[0] = out.reshape(h, w, c)


def kernel(x, Wqkv, bqkv, Wlepe, blepe, Wout, bout):
    B, C, H, W = x.shape
    nh = _NH
    hd = C // nh
    nwin = _NWIN
    rs = (H // nwin, W // nwin)
    nreg = nwin * nwin
    rsq = rs[0] * rs[1]
    scale = hd ** -0.5
    f32 = jnp.float32

    x_rt = (x.reshape(B, C, nwin, rs[0], nwin, rs[1])
             .transpose(0, 2, 4, 3, 5, 1)
             .reshape(B, nreg, rsq, C))

    wq_t = Wqkv[:C].T
    wk_t = Wqkv[C:2 * C].T
    wv_t = Wqkv[2 * C:].T
    bq = bqkv[:C].reshape(1, C)
    bk = bqkv[C:2 * C].reshape(1, C)
    bv = bqkv[2 * C:].reshape(1, C)

    full2 = pl.BlockSpec((C, C), lambda b, t: (0, 0))
    bias2 = pl.BlockSpec((1, C), lambda b, t: (0, 0))
    seq_spec = pl.BlockSpec((1, nwin, rsq, C), lambda b, t: (b, t, 0, 0))
    pool_spec = pl.BlockSpec((1, 1, nwin, C), lambda b, t: (b, t, 0, 0))
    q_rt, k_rt, v_rt, pq, pk = pl.pallas_call(
        _qkv_kernel,
        grid=(B, nwin),
        in_specs=[seq_spec, full2, full2, full2, bias2, bias2, bias2],
        out_specs=[seq_spec, seq_spec, seq_spec, pool_spec, pool_spec],
        out_shape=[
            jax.ShapeDtypeStruct((B, nreg, rsq, C), f32),
            jax.ShapeDtypeStruct((B, nreg, rsq, C), f32),
            jax.ShapeDtypeStruct((B, nreg, rsq, C), f32),
            jax.ShapeDtypeStruct((B, nwin, nwin, C), f32),
            jax.ShapeDtypeStruct((B, nwin, nwin, C), f32),
        ],
        compiler_params=pltpu.CompilerParams(
            dimension_semantics=("parallel", "parallel")),
    )(x_rt, wq_t, wk_t, wv_t, bq, bk, bv)

    q_r = pq.reshape(B, nreg, C)
    k_r = pk.reshape(B, nreg, C)
    idx = pl.pallas_call(
        _route_kernel,
        grid=(B,),
        in_specs=[pl.BlockSpec((1, nreg, C), lambda b: (b, 0, 0)),
                  pl.BlockSpec((1, nreg, C), lambda b: (b, 0, 0))],
        out_specs=pl.BlockSpec((1, nreg, _TOPK), lambda b: (b, 0, 0)),
        out_shape=jax.ShapeDtypeStruct((B, nreg, _TOPK), jnp.int32),
        compiler_params=pltpu.CompilerParams(
            dimension_semantics=("parallel",)),
    )(q_r, k_r)

    def _qmap(b, n, idx_ref):
        return (b, n, 0, 0)

    def _gmap(b, n, idx_ref, *, j):
        return (b, idx_ref[b, n, j], 0, 0)

    blk = pl.BlockSpec((1, 1, rsq, C), _qmap)
    gather_specs = [pl.BlockSpec((1, 1, rsq, C), functools.partial(_gmap, j=j))
                    for j in range(_TOPK)]
    grid_spec = pltpu.PrefetchScalarGridSpec(
        num_scalar_prefetch=1,
        grid=(B, nreg),
        in_specs=[blk] + gather_specs + gather_specs,
        out_specs=blk,
    )
    out_seq = pl.pallas_call(
        functools.partial(_attn_kernel, scale=scale, nh=nh),
        grid_spec=grid_spec,
        out_shape=jax.ShapeDtypeStruct((B, nreg, rsq, C), f32),
        compiler_params=pltpu.CompilerParams(
            dimension_semantics=("parallel", "parallel")),
    )(idx, q_rt, k_rt, k_rt, k_rt, k_rt, v_rt, v_rt, v_rt, v_rt)

    def _to_grid(t):
        return (t.reshape(B, nwin, nwin, rs[0], rs[1], C)
                 .transpose(0, 1, 3, 2, 4, 5)
                 .reshape(B, H, W, C))

    ao = _to_grid(out_seq)
    v_pad = jnp.pad(_to_grid(v_rt), ((0, 0), (1, 1), (1, 1), (0, 0)))
    wl9 = Wlepe.reshape(C, 9).T

    out_nhwc = pl.pallas_call(
        _out_kernel,
        grid=(B,),
        in_specs=[
            pl.BlockSpec((1, H, W, C), lambda b: (b, 0, 0, 0)),
            pl.BlockSpec((1, H + 2, W + 2, C), lambda b: (b, 0, 0, 0)),
            pl.BlockSpec((9, C), lambda b: (0, 0)),
            pl.BlockSpec((1, C), lambda b: (0, 0)),
            pl.BlockSpec((C, C), lambda b: (0, 0)),
            pl.BlockSpec((1, C), lambda b: (0, 0)),
        ],
        out_specs=pl.BlockSpec((1, H, W, C), lambda b: (b, 0, 0, 0)),
        out_shape=jax.ShapeDtypeStruct((B, H, W, C), f32),
        compiler_params=pltpu.CompilerParams(
            dimension_semantics=("parallel",)),
    )(ao, v_pad, wl9, blepe.reshape(1, C), Wout.T, bout.reshape(1, C))

    return out_nhwc.transpose(0, 3, 1, 2)


# speedup vs baseline: 2.9420x; 2.4616x over previous
import functools

import jax
import jax.numpy as jnp
from jax.experimental import pallas as pl
from jax.experimental.pallas import tpu as pltpu

_NH = 8
_NWIN = 7
_TOPK = 4


def _qkv_kernel(x_ref, wq_ref, wk_ref, wv_ref, bq_ref, bk_ref, bv_ref,
                q_ref, k_ref, v_ref, pq_ref, pk_ref, *, rs):
    nwin = pq_ref.shape[2]
    c = q_ref.shape[3]
    x = x_ref[0, 0]
    dims = (((0,), (0,)), ((), ()))
    q = jax.lax.dot_general(x, wq_ref[...], dims,
                            preferred_element_type=jnp.float32) + bq_ref[...]
    k = jax.lax.dot_general(x, wk_ref[...], dims,
                            preferred_element_type=jnp.float32) + bk_ref[...]
    v = jax.lax.dot_general(x, wv_ref[...], dims,
                            preferred_element_type=jnp.float32) + bv_ref[...]
    q_ref[0, 0] = q
    k_ref[0, 0] = k
    v_ref[0, 0] = v
    pq_ref[0, 0] = jnp.mean(q.reshape(rs[0], nwin, rs[1], c), axis=(0, 2))
    pk_ref[0, 0] = jnp.mean(k.reshape(rs[0], nwin, rs[1], c), axis=(0, 2))


def _route_kernel(qr_ref, kr_ref, idx_ref):
    B, nreg, _ = qr_ref.shape
    rows = []
    for b in range(B):
        rows.append(jax.lax.dot_general(
            qr_ref[b], kr_ref[b], (((1,), (1,)), ((), ())),
            preferred_element_type=jnp.float32))
    a = jnp.concatenate(rows, axis=0)
    col = jax.lax.broadcasted_iota(jnp.int32, a.shape, 1)
    picks = []
    for _ in range(_TOPK):
        m = jnp.max(a, axis=1, keepdims=True)
        cand = jnp.where(a == m, col, jnp.int32(2 ** 30))
        j = jnp.min(cand, axis=1, keepdims=True)
        picks.append(j)
        a = jnp.where(col == j, -jnp.inf, a)
    idx_ref[...] = jnp.concatenate(picks, axis=1).reshape(B, nreg, _TOPK)


def _attn_kernel(idx_ref, q_ref, k_ref, v_ref, o_ref, *, scale, nh, nwin):
    b = pl.program_id(0)
    t = pl.program_id(1)
    rw = pl.program_id(2)
    rsq = q_ref.shape[2] * q_ref.shape[4]
    c = q_ref.shape[5]
    hd = c // nh
    n = t * nwin + rw
    q = q_ref[0, 0, :, 0].reshape(rsq, c) * scale
    ks, vs = [], []
    for j in range(_TOPK):
        r = idx_ref[b, n, j]
        r1 = r // nwin
        r2 = r % nwin
        ks.append(k_ref[0, r1, :, r2].reshape(rsq, c))
        vs.append(v_ref[0, r1, :, r2].reshape(rsq, c))
    k = jnp.concatenate(ks, axis=0)
    v = jnp.concatenate(vs, axis=0)
    qt = jnp.concatenate([q] * nh, axis=0)
    row = jax.lax.broadcasted_iota(jnp.int32, (nh * rsq, c), 0)
    col = jax.lax.broadcasted_iota(jnp.int32, (nh * rsq, c), 1)
    qbd = jnp.where(row // rsq == col // hd, qt, 0.0)
    s = jax.lax.dot_general(qbd, k, (((1,), (1,)), ((), ())),
                            preferred_element_type=jnp.float32)
    m = jnp.max(s, axis=1, keepdims=True)
    e = jnp.exp(s - m)
    p = e / jnp.sum(e, axis=1, keepdims=True)
    ob = jnp.dot(p, v, preferred_element_type=jnp.float32)
    colh = col[:rsq] // hd
    acc = jnp.zeros((rsq, c), jnp.float32)
    for h in range(nh):
        acc = acc + jnp.where(colh == h, ob[h * rsq:(h + 1) * rsq], 0.0)
    o_ref[0, 0, :, 0] = acc.reshape(q_ref.shape[2], q_ref.shape[4], c)


def _out_kernel(ao_ref, vp_ref, wl_ref, bl_ref, wo_ref, bo_ref, o_ref):
    h = ao_ref.shape[1]
    w = ao_ref.shape[2]
    c = ao_ref.shape[3]
    acc = ao_ref[0]
    wl = wl_ref[...]
    for kx in range(3):
        vsh = vp_ref[0, :, kx:kx + w, :]
        for ky in range(3):
            acc = acc + vsh[ky:ky + h] * wl[ky * 3 + kx][None, None, :]
    acc = acc + bl_ref[...][None]
    t = acc.reshape(h * w, c)
    out = jnp.dot(t, wo_ref[...], preferred_element_type=jnp.float32) + bo_ref[...]
    o_ref[0] = out.reshape(h, w, c)


def kernel(x, Wqkv, bqkv, Wlepe, blepe, Wout, bout):
    B, C, H, W = x.shape
    nh = _NH
    hd = C // nh
    nwin = _NWIN
    rs = (H // nwin, W // nwin)
    nreg = nwin * nwin
    rsq = rs[0] * rs[1]
    slab = rs[0] * W
    scale = hd ** -0.5
    f32 = jnp.float32

    x3 = x.reshape(B, C, nwin, slab).transpose(0, 2, 1, 3)

    wq_t = Wqkv[:C].T
    wk_t = Wqkv[C:2 * C].T
    wv_t = Wqkv[2 * C:].T
    bq = bqkv[:C].reshape(1, C)
    bk = bqkv[C:2 * C].reshape(1, C)
    bv = bqkv[2 * C:].reshape(1, C)

    full2 = pl.BlockSpec((C, C), lambda b, t: (0, 0))
    bias2 = pl.BlockSpec((1, C), lambda b, t: (0, 0))
    seq_spec = pl.BlockSpec((1, 1, slab, C), lambda b, t: (b, t, 0, 0))
    pool_spec = pl.BlockSpec((1, 1, nwin, C), lambda b, t: (b, t, 0, 0))
    q_rt, k_rt, v_rt, pq, pk = pl.pallas_call(
        functools.partial(_qkv_kernel, rs=rs),
        grid=(B, nwin),
        in_specs=[pl.BlockSpec((1, 1, C, slab), lambda b, t: (b, t, 0, 0)),
                  full2, full2, full2, bias2, bias2, bias2],
        out_specs=[seq_spec, seq_spec, seq_spec, pool_spec, pool_spec],
        out_shape=[
            jax.ShapeDtypeStruct((B, nwin, slab, C), f32),
            jax.ShapeDtypeStruct((B, nwin, slab, C), f32),
            jax.ShapeDtypeStruct((B, nwin, slab, C), f32),
            jax.ShapeDtypeStruct((B, nwin, nwin, C), f32),
            jax.ShapeDtypeStruct((B, nwin, nwin, C), f32),
        ],
        compiler_params=pltpu.CompilerParams(
            dimension_semantics=("parallel", "parallel")),
    )(x3, wq_t, wk_t, wv_t, bq, bk, bv)

    q_r = pq.reshape(B, nreg, C)
    k_r = pk.reshape(B, nreg, C)
    idx = pl.pallas_call(
        _route_kernel,
        grid=(1,),
        in_specs=[pl.BlockSpec((B, nreg, C), lambda i: (0, 0, 0)),
                  pl.BlockSpec((B, nreg, C), lambda i: (0, 0, 0))],
        out_specs=pl.BlockSpec((B, nreg, _TOPK), lambda i: (0, 0, 0)),
        out_shape=jax.ShapeDtypeStruct((B, nreg, _TOPK), jnp.int32),
    )(q_r, k_r)

    shape6 = (B, nwin, rs[0], nwin, rs[1], C)
    q6 = q_rt.reshape(shape6)
    k6 = k_rt.reshape(shape6)
    v6 = v_rt.reshape(shape6)

    qblk = pl.BlockSpec((1, 1, rs[0], 1, rs[1], C),
                        lambda b, t, rw, idx_ref: (b, t, 0, rw, 0, 0))
    kvblk = pl.BlockSpec((1, nwin, rs[0], nwin, rs[1], C),
                         lambda b, t, rw, idx_ref: (b, 0, 0, 0, 0, 0))
    grid_spec = pltpu.PrefetchScalarGridSpec(
        num_scalar_prefetch=1,
        grid=(B, nwin, nwin),
        in_specs=[qblk, kvblk, kvblk],
        out_specs=qblk,
    )
    out_seq = pl.pallas_call(
        functools.partial(_attn_kernel, scale=scale, nh=nh, nwin=nwin),
        grid_spec=grid_spec,
        out_shape=jax.ShapeDtypeStruct(shape6, f32),
        compiler_params=pltpu.CompilerParams(
            dimension_semantics=("parallel", "arbitrary", "arbitrary")),
    )(idx, q6, k6, v6)

    ao = out_seq.reshape(B, H, W, C)
    v_pad = jnp.pad(v_rt.reshape(B, H, W, C),
                    ((0, 0), (1, 1), (1, 1), (0, 0)))
    wl9 = Wlepe.reshape(C, 9).T

    out_nhwc = pl.pallas_call(
        _out_kernel,
        grid=(B,),
        in_specs=[
            pl.BlockSpec((1, H, W, C), lambda b: (b, 0, 0, 0)),
            pl.BlockSpec((1, H + 2, W + 2, C), lambda b: (b, 0, 0, 0)),
            pl.BlockSpec((9, C), lambda b: (0, 0)),
            pl.BlockSpec((1, C), lambda b: (0, 0)),
            pl.BlockSpec((C, C), lambda b: (0, 0)),
            pl.BlockSpec((1, C), lambda b: (0, 0)),
        ],
        out_specs=pl.BlockSpec((1, H, W, C), lambda b: (b, 0, 0, 0)),
        out_shape=jax.ShapeDtypeStruct((B, H, W, C), f32),
        compiler_params=pltpu.CompilerParams(
            dimension_semantics=("parallel",)),
    )(ao, v_pad, wl9, blepe.reshape(1, C), Wout.T, bout.reshape(1, C))

    return out_nhwc.transpose(0, 3, 1, 2)


# speedup vs baseline: 3.7050x; 1.2594x over previous
import functools

import jax
import jax.numpy as jnp
from jax.experimental import pallas as pl
from jax.experimental.pallas import tpu as pltpu

_NH = 8
_NWIN = 7
_TOPK = 4


def _qkv_kernel(x_ref, wq_ref, wk_ref, wv_ref, bq_ref, bk_ref, bv_ref,
                q_ref, k_ref, v_ref, pq_ref, pk_ref, *, rs, nwin):
    c = q_ref.shape[2]
    x = x_ref[0]
    dims = (((0,), (0,)), ((), ()))
    q = jax.lax.dot_general(x, wq_ref[...], dims,
                            preferred_element_type=jnp.float32) + bq_ref[...]
    k = jax.lax.dot_general(x, wk_ref[...], dims,
                            preferred_element_type=jnp.float32) + bk_ref[...]
    v = jax.lax.dot_general(x, wv_ref[...], dims,
                            preferred_element_type=jnp.float32) + bv_ref[...]
    q_ref[0] = q
    k_ref[0] = k
    v_ref[0] = v
    pq_ref[0] = jnp.mean(
        q.reshape(nwin, rs[0], nwin, rs[1], c), axis=(1, 3)
    ).reshape(nwin * nwin, c)
    pk_ref[0] = jnp.mean(
        k.reshape(nwin, rs[0], nwin, rs[1], c), axis=(1, 3)
    ).reshape(nwin * nwin, c)


def _route_kernel(qr_ref, kr_ref, idx_ref):
    B, nreg, _ = qr_ref.shape
    rows = []
    for b in range(B):
        rows.append(jax.lax.dot_general(
            qr_ref[b], kr_ref[b], (((1,), (1,)), ((), ())),
            preferred_element_type=jnp.float32))
    a = jnp.concatenate(rows, axis=0)
    col = jax.lax.broadcasted_iota(jnp.int32, a.shape, 1)
    picks = []
    for _ in range(_TOPK):
        m = jnp.max(a, axis=1, keepdims=True)
        cand = jnp.where(a == m, col, jnp.int32(2 ** 30))
        j = jnp.min(cand, axis=1, keepdims=True)
        picks.append(j)
        a = jnp.where(col == j, -jnp.inf, a)
    idx_ref[...] = jnp.concatenate(picks, axis=1).reshape(B, nreg, _TOPK)


def _attn_kernel(idx_ref, q_ref, k_ref, v_ref, o_ref, *, scale, nh, nwin):
    b = pl.program_id(0)
    t = pl.program_id(1)
    rw = pl.program_id(2)
    rsq = q_ref.shape[2] * q_ref.shape[4]
    c = q_ref.shape[5]
    hd = c // nh
    n = t * nwin + rw
    q = q_ref[0, 0, :, 0].reshape(rsq, c) * scale
    ks, vs = [], []
    for j in range(_TOPK):
        r = idx_ref[b, n, j]
        r1 = r // nwin
        r2 = r % nwin
        ks.append(k_ref[0, r1, :, r2].reshape(rsq, c))
        vs.append(v_ref[0, r1, :, r2].reshape(rsq, c))
    k = jnp.concatenate(ks, axis=0)
    v = jnp.concatenate(vs, axis=0)
    qt = jnp.concatenate([q] * nh, axis=0)
    row = jax.lax.broadcasted_iota(jnp.int32, (nh * rsq, c), 0)
    col = jax.lax.broadcasted_iota(jnp.int32, (nh * rsq, c), 1)
    qbd = jnp.where(row // rsq == col // hd, qt, 0.0)
    s = jax.lax.dot_general(qbd, k, (((1,), (1,)), ((), ())),
                            preferred_element_type=jnp.float32)
    m = jnp.max(s, axis=1, keepdims=True)
    e = jnp.exp(s - m)
    p = e / jnp.sum(e, axis=1, keepdims=True)
    ob = jnp.dot(p, v, preferred_element_type=jnp.float32)
    colh = col[:rsq] // hd
    acc = jnp.zeros((rsq, c), jnp.float32)
    for h in range(nh):
        acc = acc + jnp.where(colh == h, ob[h * rsq:(h + 1) * rsq], 0.0)
    o_ref[0, 0, :, 0] = acc.reshape(q_ref.shape[2], q_ref.shape[4], c)


def _out_kernel(ao_ref, v_ref, wl_ref, bl_ref, wo_ref, bo_ref, o_ref):
    h = ao_ref.shape[1]
    w = ao_ref.shape[2]
    c = ao_ref.shape[3]
    acc = ao_ref[0]
    v = v_ref[0]
    wl = wl_ref[...]
    zx = jnp.zeros((h, 1, c), jnp.float32)
    zy = jnp.zeros((1, w, c), jnp.float32)
    for kx in range(3):
        if kx == 0:
            vx = jnp.concatenate([zx, v[:, :w - 1, :]], axis=1)
        elif kx == 1:
            vx = v
        else:
            vx = jnp.concatenate([v[:, 1:, :], zx], axis=1)
        for ky in range(3):
            if ky == 0:
                vs = jnp.concatenate([zy, vx[:h - 1]], axis=0)
            elif ky == 1:
                vs = vx
            else:
                vs = jnp.concatenate([vx[1:], zy], axis=0)
            acc = acc + vs * wl[ky * 3 + kx][None, None, :]
    acc = acc + bl_ref[...][None]
    t = acc.reshape(h * w, c)
    out = jax.lax.dot_general(wo_ref[...], t, (((1,), (1,)), ((), ())),
                              preferred_element_type=jnp.float32)
    o_ref[0] = out + bo_ref[...]


def kernel(x, Wqkv, bqkv, Wlepe, blepe, Wout, bout):
    B, C, H, W = x.shape
    nh = _NH
    hd = C // nh
    nwin = _NWIN
    rs = (H // nwin, W // nwin)
    nreg = nwin * nwin
    hw = H * W
    scale = hd ** -0.5
    f32 = jnp.float32

    x3 = x.reshape(B, C, hw)

    wq_t = Wqkv[:C].T
    wk_t = Wqkv[C:2 * C].T
    wv_t = Wqkv[2 * C:].T
    bq = bqkv[:C].reshape(1, C)
    bk = bqkv[C:2 * C].reshape(1, C)
    bv = bqkv[2 * C:].reshape(1, C)

    full2 = pl.BlockSpec((C, C), lambda b: (0, 0))
    bias2 = pl.BlockSpec((1, C), lambda b: (0, 0))
    seq_spec = pl.BlockSpec((1, hw, C), lambda b: (b, 0, 0))
    pool_spec = pl.BlockSpec((1, nreg, C), lambda b: (b, 0, 0))
    q_rt, k_rt, v_rt, pq, pk = pl.pallas_call(
        functools.partial(_qkv_kernel, rs=rs, nwin=nwin),
        grid=(B,),
        in_specs=[pl.BlockSpec((1, C, hw), lambda b: (b, 0, 0)),
                  full2, full2, full2, bias2, bias2, bias2],
        out_specs=[seq_spec, seq_spec, seq_spec, pool_spec, pool_spec],
        out_shape=[
            jax.ShapeDtypeStruct((B, hw, C), f32),
            jax.ShapeDtypeStruct((B, hw, C), f32),
            jax.ShapeDtypeStruct((B, hw, C), f32),
            jax.ShapeDtypeStruct((B, nreg, C), f32),
            jax.ShapeDtypeStruct((B, nreg, C), f32),
        ],
        compiler_params=pltpu.CompilerParams(
            dimension_semantics=("parallel",)),
    )(x3, wq_t, wk_t, wv_t, bq, bk, bv)

    idx = pl.pallas_call(
        _route_kernel,
        grid=(1,),
        in_specs=[pl.BlockSpec((B, nreg, C), lambda i: (0, 0, 0)),
                  pl.BlockSpec((B, nreg, C), lambda i: (0, 0, 0))],
        out_specs=pl.BlockSpec((B, nreg, _TOPK), lambda i: (0, 0, 0)),
        out_shape=jax.ShapeDtypeStruct((B, nreg, _TOPK), jnp.int32),
    )(pq, pk)

    shape6 = (B, nwin, rs[0], nwin, rs[1], C)
    q6 = q_rt.reshape(shape6)
    k6 = k_rt.reshape(shape6)
    v6 = v_rt.reshape(shape6)

    qblk = pl.BlockSpec((1, 1, rs[0], 1, rs[1], C),
                        lambda b, t, rw, idx_ref: (b, t, 0, rw, 0, 0))
    kvblk = pl.BlockSpec((1, nwin, rs[0], nwin, rs[1], C),
                         lambda b, t, rw, idx_ref: (b, 0, 0, 0, 0, 0))
    grid_spec = pltpu.PrefetchScalarGridSpec(
        num_scalar_prefetch=1,
        grid=(B, nwin, nwin),
        in_specs=[qblk, kvblk, kvblk],
        out_specs=qblk,
    )
    out_seq = pl.pallas_call(
        functools.partial(_attn_kernel, scale=scale, nh=nh, nwin=nwin),
        grid_spec=grid_spec,
        out_shape=jax.ShapeDtypeStruct(shape6, f32),
        compiler_params=pltpu.CompilerParams(
            dimension_semantics=("parallel", "arbitrary", "arbitrary")),
    )(idx, q6, k6, v6)

    ao = out_seq.reshape(B, H, W, C)
    v_img = v_rt.reshape(B, H, W, C)
    wl9 = Wlepe.reshape(C, 9).T

    out_cm = pl.pallas_call(
        _out_kernel,
        grid=(B,),
        in_specs=[
            pl.BlockSpec((1, H, W, C), lambda b: (b, 0, 0, 0)),
            pl.BlockSpec((1, H, W, C), lambda b: (b, 0, 0, 0)),
            pl.BlockSpec((9, C), lambda b: (0, 0)),
            pl.BlockSpec((1, C), lambda b: (0, 0)),
            pl.BlockSpec((C, C), lambda b: (0, 0)),
            pl.BlockSpec((C, 1), lambda b: (0, 0)),
        ],
        out_specs=pl.BlockSpec((1, C, hw), lambda b: (b, 0, 0)),
        out_shape=jax.ShapeDtypeStruct((B, C, hw), f32),
        compiler_params=pltpu.CompilerParams(
            dimension_semantics=("parallel",)),
    )(ao, v_img, wl9, blepe.reshape(1, C), Wout, bout.reshape(C, 1))

    return out_cm.reshape(B, C, H, W)


# speedup vs baseline: 6.8175x; 1.8401x over previous
import functools

import jax
import jax.numpy as jnp
from jax.experimental import pallas as pl
from jax.experimental.pallas import tpu as pltpu

_NH = 8
_NWIN = 7
_TOPK = 4


def _qkv_kernel(x_ref, wq_ref, wk_ref, wv_ref, bq_ref, bk_ref, bv_ref,
                q_ref, k_ref, v_ref, pq_ref, pk_ref, *, rs, nwin):
    c = q_ref.shape[2]
    x = x_ref[0]
    dims = (((0,), (0,)), ((), ()))
    q = jax.lax.dot_general(x, wq_ref[...], dims,
                            preferred_element_type=jnp.float32) + bq_ref[...]
    k = jax.lax.dot_general(x, wk_ref[...], dims,
                            preferred_element_type=jnp.float32) + bk_ref[...]
    v = jax.lax.dot_general(x, wv_ref[...], dims,
                            preferred_element_type=jnp.float32) + bv_ref[...]
    q_ref[0] = q
    k_ref[0] = k
    v_ref[0] = v
    pq_ref[0] = jnp.mean(
        q.reshape(nwin, rs[0], nwin, rs[1], c), axis=(1, 3)
    ).reshape(nwin * nwin, c)
    pk_ref[0] = jnp.mean(
        k.reshape(nwin, rs[0], nwin, rs[1], c), axis=(1, 3)
    ).reshape(nwin * nwin, c)


def _route_kernel(qr_ref, kr_ref, idx_ref):
    B, nreg, _ = qr_ref.shape
    rows = []
    for b in range(B):
        rows.append(jax.lax.dot_general(
            qr_ref[b], kr_ref[b], (((1,), (1,)), ((), ())),
            preferred_element_type=jnp.float32))
    a = jnp.concatenate(rows, axis=0)
    col = jax.lax.broadcasted_iota(jnp.int32, a.shape, 1)
    picks = []
    for _ in range(_TOPK):
        m = jnp.max(a, axis=1, keepdims=True)
        cand = jnp.where(a == m, col, jnp.int32(2 ** 30))
        j = jnp.min(cand, axis=1, keepdims=True)
        picks.append(j)
        a = jnp.where(col == j, -jnp.inf, a)
    idx_ref[...] = jnp.concatenate(picks, axis=1).reshape(B, nreg, _TOPK)


def _attn_kernel(idx_ref, q_ref, k_ref, v_ref, o_ref, *, scale, nh, nwin):
    b = pl.program_id(0)
    t = pl.program_id(1)
    rsq = q_ref.shape[2] * q_ref.shape[4]
    c = q_ref.shape[5]
    hd = c // nh
    row = jax.lax.broadcasted_iota(jnp.int32, (nh * rsq, c), 0)
    col = jax.lax.broadcasted_iota(jnp.int32, (nh * rsq, c), 1)
    bd_mask = row // rsq == col // hd
    colh = col[:rsq] // hd
    for rw in range(nwin):
        n = t * nwin + rw
        q = q_ref[0, 0, :, rw].reshape(rsq, c) * scale
        ks, vs = [], []
        for j in range(_TOPK):
            r = idx_ref[b, n, j]
            r1 = r // nwin
            r2 = r % nwin
            ks.append(k_ref[0, r1, :, r2].reshape(rsq, c))
            vs.append(v_ref[0, r1, :, r2].reshape(rsq, c))
        k = jnp.concatenate(ks, axis=0)
        v = jnp.concatenate(vs, axis=0)
        qt = jnp.concatenate([q] * nh, axis=0)
        qbd = jnp.where(bd_mask, qt, 0.0)
        s = jax.lax.dot_general(qbd, k, (((1,), (1,)), ((), ())),
                                preferred_element_type=jnp.float32)
        m = jnp.max(s, axis=1, keepdims=True)
        e = jnp.exp(s - m)
        p = e / jnp.sum(e, axis=1, keepdims=True)
        ob = jnp.dot(p, v, preferred_element_type=jnp.float32)
        acc = jnp.zeros((rsq, c), jnp.float32)
        for h in range(nh):
            acc = acc + jnp.where(colh == h, ob[h * rsq:(h + 1) * rsq], 0.0)
        o_ref[0, 0, :, rw] = acc.reshape(q_ref.shape[2], q_ref.shape[4], c)


def _out_kernel(ao_ref, v_ref, wl_ref, bl_ref, wo_ref, bo_ref, o_ref):
    h = ao_ref.shape[1]
    w = ao_ref.shape[2]
    c = ao_ref.shape[3]
    acc = ao_ref[0]
    v = v_ref[0]
    wl = wl_ref[...]
    zx = jnp.zeros((h, 1, c), jnp.float32)
    zy = jnp.zeros((1, w, c), jnp.float32)
    for kx in range(3):
        if kx == 0:
            vx = jnp.concatenate([zx, v[:, :w - 1, :]], axis=1)
        elif kx == 1:
            vx = v
        else:
            vx = jnp.concatenate([v[:, 1:, :], zx], axis=1)
        for ky in range(3):
            if ky == 0:
                vs = jnp.concatenate([zy, vx[:h - 1]], axis=0)
            elif ky == 1:
                vs = vx
            else:
                vs = jnp.concatenate([vx[1:], zy], axis=0)
            acc = acc + vs * wl[ky * 3 + kx][None, None, :]
    acc = acc + bl_ref[...][None]
    t = acc.reshape(h * w, c)
    out = jax.lax.dot_general(wo_ref[...], t, (((1,), (1,)), ((), ())),
                              preferred_element_type=jnp.float32)
    o_ref[0] = out + bo_ref[...]


def kernel(x, Wqkv, bqkv, Wlepe, blepe, Wout, bout):
    B, C, H, W = x.shape
    nh = _NH
    hd = C // nh
    nwin = _NWIN
    rs = (H // nwin, W // nwin)
    nreg = nwin * nwin
    hw = H * W
    scale = hd ** -0.5
    f32 = jnp.float32

    x3 = x.reshape(B, C, hw)

    wq_t = Wqkv[:C].T
    wk_t = Wqkv[C:2 * C].T
    wv_t = Wqkv[2 * C:].T
    bq = bqkv[:C].reshape(1, C)
    bk = bqkv[C:2 * C].reshape(1, C)
    bv = bqkv[2 * C:].reshape(1, C)

    full2 = pl.BlockSpec((C, C), lambda b: (0, 0))
    bias2 = pl.BlockSpec((1, C), lambda b: (0, 0))
    seq_spec = pl.BlockSpec((1, hw, C), lambda b: (b, 0, 0))
    pool_spec = pl.BlockSpec((1, nreg, C), lambda b: (b, 0, 0))
    q_rt, k_rt, v_rt, pq, pk = pl.pallas_call(
        functools.partial(_qkv_kernel, rs=rs, nwin=nwin),
        grid=(B,),
        in_specs=[pl.BlockSpec((1, C, hw), lambda b: (b, 0, 0)),
                  full2, full2, full2, bias2, bias2, bias2],
        out_specs=[seq_spec, seq_spec, seq_spec, pool_spec, pool_spec],
        out_shape=[
            jax.ShapeDtypeStruct((B, hw, C), f32),
            jax.ShapeDtypeStruct((B, hw, C), f32),
            jax.ShapeDtypeStruct((B, hw, C), f32),
            jax.ShapeDtypeStruct((B, nreg, C), f32),
            jax.ShapeDtypeStruct((B, nreg, C), f32),
        ],
        compiler_params=pltpu.CompilerParams(
            dimension_semantics=("parallel",)),
    )(x3, wq_t, wk_t, wv_t, bq, bk, bv)

    idx = pl.pallas_call(
        _route_kernel,
        grid=(1,),
        in_specs=[pl.BlockSpec((B, nreg, C), lambda i: (0, 0, 0)),
                  pl.BlockSpec((B, nreg, C), lambda i: (0, 0, 0))],
        out_specs=pl.BlockSpec((B, nreg, _TOPK), lambda i: (0, 0, 0)),
        out_shape=jax.ShapeDtypeStruct((B, nreg, _TOPK), jnp.int32),
    )(pq, pk)

    shape6 = (B, nwin, rs[0], nwin, rs[1], C)
    q6 = q_rt.reshape(shape6)
    k6 = k_rt.reshape(shape6)
    v6 = v_rt.reshape(shape6)

    qblk = pl.BlockSpec((1, 1, rs[0], nwin, rs[1], C),
                        lambda b, t, idx_ref: (b, t, 0, 0, 0, 0))
    kvblk = pl.BlockSpec((1, nwin, rs[0], nwin, rs[1], C),
                         lambda b, t, idx_ref: (b, 0, 0, 0, 0, 0))
    grid_spec = pltpu.PrefetchScalarGridSpec(
        num_scalar_prefetch=1,
        grid=(B, nwin),
        in_specs=[qblk, kvblk, kvblk],
        out_specs=qblk,
    )
    out_seq = pl.pallas_call(
        functools.partial(_attn_kernel, scale=scale, nh=nh, nwin=nwin),
        grid_spec=grid_spec,
        out_shape=jax.ShapeDtypeStruct(shape6, f32),
        compiler_params=pltpu.CompilerParams(
            dimension_semantics=("parallel", "arbitrary")),
    )(idx, q6, k6, v6)

    ao = out_seq.reshape(B, H, W, C)
    v_img = v_rt.reshape(B, H, W, C)
    wl9 = Wlepe.reshape(C, 9).T

    out_cm = pl.pallas_call(
        _out_kernel,
        grid=(B,),
        in_specs=[
            pl.BlockSpec((1, H, W, C), lambda b: (b, 0, 0, 0)),
            pl.BlockSpec((1, H, W, C), lambda b: (b, 0, 0, 0)),
            pl.BlockSpec((9, C), lambda b: (0, 0)),
            pl.BlockSpec((1, C), lambda b: (0, 0)),
            pl.BlockSpec((C, C), lambda b: (0, 0)),
            pl.BlockSpec((C, 1), lambda b: (0, 0)),
        ],
        out_specs=pl.BlockSpec((1, C, hw), lambda b: (b, 0, 0)),
        out_shape=jax.ShapeDtypeStruct((B, C, hw), f32),
        compiler_params=pltpu.CompilerParams(
            dimension_semantics=("parallel",)),
    )(ao, v_img, wl9, blepe.reshape(1, C), Wout, bout.reshape(C, 1))

    return out_cm.reshape(B, C, H, W)


# speedup vs baseline: 8.0696x; 1.1837x over previous
import functools

import jax
import jax.numpy as jnp
from jax.experimental import pallas as pl
from jax.experimental.pallas import tpu as pltpu

_NH = 8
_NWIN = 7
_TOPK = 4


def _route_kernel(x_ref, wq_ref, wk_ref, bq_ref, bk_ref, idx_ref, *, rs):
    c, hw = x_ref.shape[1], x_ref.shape[2]
    nwin = _NWIN
    nreg = nwin * nwin
    x = x_ref[0]
    dims = (((0,), (0,)), ((), ()))
    qf = jax.lax.dot_general(x, wq_ref[...], dims,
                             preferred_element_type=jnp.float32) + bq_ref[...]
    kf = jax.lax.dot_general(x, wk_ref[...], dims,
                             preferred_element_type=jnp.float32) + bk_ref[...]
    shp5 = (nwin, rs[0], nwin, rs[1], c)
    qp = jnp.mean(qf.reshape(shp5), axis=(1, 3)).reshape(nreg, c)
    kp = jnp.mean(kf.reshape(shp5), axis=(1, 3)).reshape(nreg, c)
    a = jax.lax.dot_general(qp, kp, (((1,), (1,)), ((), ())),
                            preferred_element_type=jnp.float32)
    col = jax.lax.broadcasted_iota(jnp.int32, a.shape, 1)
    picks = []
    for _ in range(_TOPK):
        m = jnp.max(a, axis=1, keepdims=True)
        cand = jnp.where(a == m, col, jnp.int32(2 ** 30))
        j = jnp.min(cand, axis=1, keepdims=True)
        picks.append(j)
        a = jnp.where(col == j, -jnp.inf, a)
    idx_ref[0] = jnp.concatenate(picks, axis=1)


def _fused_kernel(idx_ref, x_ref, wq_ref, wk_ref, wv_ref, bq_ref, bk_ref,
                  bv_ref, wl_ref, bl_ref, wo_ref, bo_ref, o_ref,
                  q_scr, k_scr, v_scr, ao_scr, *, scale, nh, rs):
    b = pl.program_id(0)
    nwin = _NWIN
    c = x_ref.shape[1]
    hd = c // nh
    rsq = rs[0] * rs[1]
    hh = nwin * rs[0]
    ww = nwin * rs[1]
    x = x_ref[0]
    dims = (((0,), (0,)), ((), ()))
    q = jax.lax.dot_general(x, wq_ref[...], dims,
                            preferred_element_type=jnp.float32) + bq_ref[...]
    k = jax.lax.dot_general(x, wk_ref[...], dims,
                            preferred_element_type=jnp.float32) + bk_ref[...]
    v = jax.lax.dot_general(x, wv_ref[...], dims,
                            preferred_element_type=jnp.float32) + bv_ref[...]
    shp5 = (nwin, rs[0], nwin, rs[1], c)
    q_scr[...] = (q * scale).reshape(shp5)
    k_scr[...] = k.reshape(shp5)
    v_scr[...] = v.reshape(shp5)

    row = jax.lax.broadcasted_iota(jnp.int32, (nh * rsq, c), 0)
    col = jax.lax.broadcasted_iota(jnp.int32, (nh * rsq, c), 1)
    bd_mask = row // rsq == col // hd
    colh = col[:rsq] // hd
    for n in range(nwin * nwin):
        t, rw = n // nwin, n % nwin
        qn = q_scr[t, :, rw].reshape(rsq, c)
        ks, vs = [], []
        for j in range(_TOPK):
            r = idx_ref[b, n, j]
            r1 = r // nwin
            r2 = r % nwin
            ks.append(k_scr[r1, :, r2].reshape(rsq, c))
            vs.append(v_scr[r1, :, r2].reshape(rsq, c))
        kg = jnp.concatenate(ks, axis=0)
        vg = jnp.concatenate(vs, axis=0)
        qt = jnp.concatenate([qn] * nh, axis=0)
        qbd = jnp.where(bd_mask, qt, 0.0)
        s = jax.lax.dot_general(qbd, kg, (((1,), (1,)), ((), ())),
                                preferred_element_type=jnp.float32)
        m = jnp.max(s, axis=1, keepdims=True)
        e = jnp.exp(s - m)
        p = e / jnp.sum(e, axis=1, keepdims=True)
        ob = jnp.dot(p, vg, preferred_element_type=jnp.float32)
        acc = jnp.zeros((rsq, c), jnp.float32)
        for h in range(nh):
            acc = acc + jnp.where(colh == h, ob[h * rsq:(h + 1) * rsq], 0.0)
        ao_scr[t, :, rw] = acc.reshape(rs[0], rs[1], c)

    acc = ao_scr[...].reshape(hh, ww, c)
    vimg = v_scr[...].reshape(hh, ww, c)
    wl = wl_ref[...]
    zx = jnp.zeros((hh, 1, c), jnp.float32)
    zy = jnp.zeros((1, ww, c), jnp.float32)
    for kx in range(3):
        if kx == 0:
            vx = jnp.concatenate([zx, vimg[:, :ww - 1, :]], axis=1)
        elif kx == 1:
            vx = vimg
        else:
            vx = jnp.concatenate([vimg[:, 1:, :], zx], axis=1)
        for ky in range(3):
            if ky == 0:
                vsh = jnp.concatenate([zy, vx[:hh - 1]], axis=0)
            elif ky == 1:
                vsh = vx
            else:
                vsh = jnp.concatenate([vx[1:], zy], axis=0)
            acc = acc + vsh * wl[ky * 3 + kx][None, None, :]
    acc = acc + bl_ref[...][None]
    t2 = acc.reshape(hh * ww, c)
    out = jax.lax.dot_general(wo_ref[...], t2, (((1,), (1,)), ((), ())),
                              preferred_element_type=jnp.float32)
    o_ref[0] = out + bo_ref[...]


def kernel(x, Wqkv, bqkv, Wlepe, blepe, Wout, bout):
    B, C, H, W = x.shape
    nh = _NH
    hd = C // nh
    nwin = _NWIN
    rs = (H // nwin, W // nwin)
    nreg = nwin * nwin
    hw = H * W
    scale = hd ** -0.5
    f32 = jnp.float32

    x3 = x.reshape(B, C, hw)

    wq_t = Wqkv[:C].T
    wk_t = Wqkv[C:2 * C].T
    wv_t = Wqkv[2 * C:].T
    bq = bqkv[:C].reshape(1, C)
    bk = bqkv[C:2 * C].reshape(1, C)
    bv = bqkv[2 * C:].reshape(1, C)
    wl9 = Wlepe.reshape(C, 9).T

    xblk = pl.BlockSpec((1, C, hw), lambda b: (b, 0, 0))
    full2 = pl.BlockSpec((C, C), lambda b: (0, 0))
    bias2 = pl.BlockSpec((1, C), lambda b: (0, 0))
    idx = pl.pallas_call(
        functools.partial(_route_kernel, rs=rs),
        grid=(B,),
        in_specs=[xblk, full2, full2, bias2, bias2],
        out_specs=pl.BlockSpec((1, nreg, _TOPK), lambda b: (b, 0, 0)),
        out_shape=jax.ShapeDtypeStruct((B, nreg, _TOPK), jnp.int32),
        compiler_params=pltpu.CompilerParams(
            dimension_semantics=("parallel",)),
    )(x3, wq_t, wk_t, bq, bk)

    xblk2 = pl.BlockSpec((1, C, hw), lambda b, idx_ref: (b, 0, 0))
    full2p = pl.BlockSpec((C, C), lambda b, idx_ref: (0, 0))
    bias2p = pl.BlockSpec((1, C), lambda b, idx_ref: (0, 0))
    shp5 = (nwin, rs[0], nwin, rs[1], C)
    grid_spec = pltpu.PrefetchScalarGridSpec(
        num_scalar_prefetch=1,
        grid=(B,),
        in_specs=[xblk2, full2p, full2p, full2p, bias2p, bias2p, bias2p,
                  pl.BlockSpec((9, C), lambda b, idx_ref: (0, 0)),
                  bias2p, full2p,
                  pl.BlockSpec((C, 1), lambda b, idx_ref: (0, 0))],
        out_specs=pl.BlockSpec((1, C, hw), lambda b, idx_ref: (b, 0, 0)),
        scratch_shapes=[pltpu.VMEM(shp5, f32), pltpu.VMEM(shp5, f32),
                        pltpu.VMEM(shp5, f32), pltpu.VMEM(shp5, f32)],
    )
    out_cm = pl.pallas_call(
        functools.partial(_fused_kernel, scale=scale, nh=nh, rs=rs),
        grid_spec=grid_spec,
        out_shape=jax.ShapeDtypeStruct((B, C, hw), f32),
        compiler_params=pltpu.CompilerParams(
            dimension_semantics=("arbitrary",)),
    )(idx, x3, wq_t, wk_t, wv_t, bq, bk, bv, wl9, blepe.reshape(1, C),
      Wout, bout.reshape(C, 1))

    return out_cm.reshape(B, C, H, W)
